# Initial kernel scaffold; baseline (speedup 1.0000x reference)
import jax
import jax.numpy as jnp
from jax.experimental import pallas as pl


def kernel(x, edge_index, mask, adjacency, W1, a_src1, a_dst1, b1, g1, be1, W2, a_src2, a_dst2, b2, g2, be2, Wf, bf):
    raise NotImplementedError("write your pallas kernel here")


# speedup vs baseline: 26.0737x; 26.0737x over previous
import functools

import jax
import jax.numpy as jnp
from jax import lax
from jax.experimental import pallas as pl
from jax.experimental.pallas import tpu as pltpu
from jax.experimental.pallas import tpu_sc as plsc

NN = 2560
EIN = 81920
ET = EIN + NN
HH = 4
CH = 128
F = HH * CH
NW = 32
EPW = ET // NW
CHUNK = 48
SB = CHUNK // 16
NCH = EPW // CHUNK
RPT = NN // 16


def _mp_body(h_hbm, as_hbm, ad_hbm, s2_hbm, d2_hbm, acc_hbm, den_hbm,
             sidx2_v, didx2_v, rows_v, dstg_v, astg_s, astg_d,
             zrow_v, acc_sh, den_sh, sem, sem2):
    cid = lax.axis_index("c")
    sid = lax.axis_index("s")
    wid = cid * 16 + sid
    z16 = jnp.zeros((16,), jnp.float32)

    pltpu.sync_copy(s2_hbm.at[wid], sidx2_v)
    pltpu.sync_copy(d2_hbm.at[wid], didx2_v)

    for j in range(F // 16):
        zrow_v[pl.ds(j * 16, 16)] = z16
    for j in range(CHUNK):
        dstg_v[j, :] = z16

    def _zero(r, _):
        pltpu.sync_copy(zrow_v, acc_sh.at[sid * RPT + r])
        return 0
    lax.fori_loop(0, RPT, _zero, 0)

    def _zerod(r, _):
        pltpu.sync_copy(dstg_v.at[pl.ds(0, 16)],
                        den_sh.at[pl.ds(sid * RPT + r * 16, 16)])
        return 0
    lax.fori_loop(0, RPT // 16, _zerod, 0)

    plsc.subcore_barrier()

    def _chunk(c, _):
        cp1 = pltpu.async_copy(h_hbm.at[sidx2_v.at[c]], rows_v, sem)
        cp2 = pltpu.async_copy(as_hbm.at[sidx2_v.at[c]], astg_s, sem2)
        cp3 = pltpu.async_copy(ad_hbm.at[didx2_v.at[c]], astg_d, sem2)
        cp2.wait()
        cp3.wait()

        def _alpha(i, _):
            e = astg_s[i, :] + astg_d[i, :]
            e = jnp.maximum(e, e * 0.2)
            dstg_v[i, :] = jnp.exp(e)
            return 0
        lax.fori_loop(0, CHUNK, _alpha, 0)
        cp1.wait()

        def _scale(i, _):
            arow = dstg_v[i, :]
            for h in range(HH):
                m = arow[jnp.full((16,), h, jnp.int32)]
                for j in range(CH // 16):
                    sl = pl.ds(h * CH + j * 16, 16)
                    rows_v[i, sl] = rows_v[i, sl] * m
            return 0
        lax.fori_loop(0, CHUNK, _scale, 0)
        pltpu.sync_copy(rows_v, acc_sh.at[didx2_v.at[c]], add=True)
        pltpu.sync_copy(dstg_v, den_sh.at[didx2_v.at[c]], add=True)
        return 0

    lax.fori_loop(0, NCH, _chunk, 0)

    plsc.subcore_barrier()

    def _wb(r, _):
        row0 = sid * RPT + r * 16
        pltpu.sync_copy(acc_sh.at[pl.ds(row0, 16)],
                        acc_hbm.at[pl.ds(cid * NN + row0, 16)])
        pltpu.sync_copy(den_sh.at[pl.ds(row0, 16)],
                        den_hbm.at[pl.ds(cid * NN + row0, 16)])
        return 0
    lax.fori_loop(0, RPT // 16, _wb, 0)


_mp_call = functools.partial(
    pl.kernel,
    out_type=[jax.ShapeDtypeStruct((2 * NN, F), jnp.float32),
              jax.ShapeDtypeStruct((2 * NN, 16), jnp.float32)],
    mesh=plsc.VectorSubcoreMesh(core_axis_name="c", subcore_axis_name="s"),
    compiler_params=pltpu.CompilerParams(
        needs_layout_passes=False, use_tc_tiling_on_sc=False),
    scratch_types=[
        pltpu.VMEM((NCH, CHUNK), jnp.int32),
        pltpu.VMEM((NCH, CHUNK), jnp.int32),
        pltpu.VMEM((CHUNK, F), jnp.float32),
        pltpu.VMEM((CHUNK, 16), jnp.float32),
        pltpu.VMEM((CHUNK, 16), jnp.float32),
        pltpu.VMEM((CHUNK, 16), jnp.float32),
        pltpu.VMEM((F,), jnp.float32),
        pltpu.VMEM_SHARED((NN, F), jnp.float32),
        pltpu.VMEM_SHARED((NN, 16), jnp.float32),
        pltpu.SemaphoreType.DMA,
        pltpu.SemaphoreType.DMA,
    ],
)(_mp_body)


def _tc_a_body(x_ref, w_ref, afs_ref, afd_ref, h_ref, as_ref, ad_ref):
    h = jnp.dot(x_ref[...], w_ref[...], preferred_element_type=jnp.float32)
    h_ref[...] = h
    as_ref[...] = jnp.dot(h, afs_ref[...], preferred_element_type=jnp.float32)
    ad_ref[...] = jnp.dot(h, afd_ref[...], preferred_element_type=jnp.float32)


def _tc_mid_body(acc_ref, den_ref, b_ref, g_ref, be_ref, w_ref, afs_ref,
                 afd_ref, h_ref, as_ref, ad_ref):
    den = den_ref[0:NN, :] + den_ref[NN:2 * NN, :]
    parts = []
    for h in range(HH):
        a = acc_ref[0:NN, h * CH:(h + 1) * CH] + acc_ref[NN:2 * NN, h * CH:(h + 1) * CH]
        parts.append(a / den[:, h:h + 1])
    hc = jnp.concatenate(parts, axis=1) + b_ref[...][None, :]
    mu = jnp.mean(hc, axis=0)
    var = jnp.mean((hc - mu[None, :]) ** 2, axis=0)
    xn = (hc - mu[None, :]) * lax.rsqrt(var + 1e-5)[None, :]
    xn = xn * g_ref[...][None, :] + be_ref[...][None, :]
    r = jnp.maximum(xn, 0.0)
    h2 = jnp.dot(r, w_ref[...], preferred_element_type=jnp.float32)
    h_ref[...] = h2
    as_ref[...] = jnp.dot(h2, afs_ref[...], preferred_element_type=jnp.float32)
    ad_ref[...] = jnp.dot(h2, afd_ref[...], preferred_element_type=jnp.float32)


def _tc_out_body(acc_ref, den_ref, b_ref, g_ref, be_ref, wf_ref, bf_ref,
                 mask_ref, adj_ref, o_ref):
    den = den_ref[0:NN, :] + den_ref[NN:2 * NN, :]
    parts = []
    for h in range(HH):
        a = acc_ref[0:NN, h * CH:(h + 1) * CH] + acc_ref[NN:2 * NN, h * CH:(h + 1) * CH]
        parts.append(a / den[:, h:h + 1])
    hc = jnp.concatenate(parts, axis=1) + b_ref[...][None, :]
    mu = jnp.mean(hc, axis=0)
    var = jnp.mean((hc - mu[None, :]) ** 2, axis=0)
    xn = (hc - mu[None, :]) * lax.rsqrt(var + 1e-5)[None, :]
    xn = xn * g_ref[...][None, :] + be_ref[...][None, :]
    r = jnp.maximum(xn, 0.0)
    o = jnp.dot(r, wf_ref[...], preferred_element_type=jnp.float32) + bf_ref[...][None, :]
    o = o * mask_ref[...][:, None] + adj_ref[...]
    o_ref[...] = o


def _build_af(a_src, a_dst, ch):
    eye = jnp.eye(HH, dtype=jnp.float32)
    bs = jnp.einsum("hc,hk->hck", a_src, eye).reshape(HH * ch, HH)
    bd = jnp.einsum("hc,hk->hck", a_dst, eye).reshape(HH * ch, HH)
    pad = jnp.zeros((HH * ch, 16 - HH), jnp.float32)
    return (jnp.concatenate([bs, pad], axis=1),
            jnp.concatenate([bd, pad], axis=1))


def kernel(x, edge_index, mask, adjacency, W1, a_src1, a_dst1, b1, g1, be1,
           W2, a_src2, a_dst2, b2, g2, be2, Wf, bf):
    loop = jnp.arange(NN, dtype=edge_index.dtype)
    s_idx = jnp.concatenate([edge_index[0], loop]).astype(jnp.int32)
    d_idx = jnp.concatenate([edge_index[1], loop]).astype(jnp.int32)
    s_idx2 = s_idx.reshape(NW, NCH, CHUNK)
    d_idx2 = d_idx.reshape(NW, NCH, CHUNK)
    afs1, afd1 = _build_af(a_src1, a_dst1, CH)
    afs2, afd2 = _build_af(a_src2, a_dst2, CH)

    h1, as1, ad1 = pl.pallas_call(
        _tc_a_body,
        out_shape=[jax.ShapeDtypeStruct((NN, F), jnp.float32),
                   jax.ShapeDtypeStruct((NN, 16), jnp.float32),
                   jax.ShapeDtypeStruct((NN, 16), jnp.float32)],
    )(x, W1, afs1, afd1)

    acc1, den1 = _mp_call(h1, as1, ad1, s_idx2, d_idx2)

    h2, as2, ad2 = pl.pallas_call(
        _tc_mid_body,
        out_shape=[jax.ShapeDtypeStruct((NN, F), jnp.float32),
                   jax.ShapeDtypeStruct((NN, 16), jnp.float32),
                   jax.ShapeDtypeStruct((NN, 16), jnp.float32)],
    )(acc1, den1[:, :4], b1, g1, be1, W2, afs2, afd2)

    acc2, den2 = _mp_call(h2, as2, ad2, s_idx2, d_idx2)

    o = pl.pallas_call(
        _tc_out_body,
        out_shape=jax.ShapeDtypeStruct((NN, 40), jnp.float32),
    )(acc2, den2[:, :4], b2, g2, be2, Wf, bf, mask, adjacency)

    return o.reshape(64, 40, 40)
